# baseline (speedup 1.0000x reference)
import functools

import jax
import jax.numpy as jnp
from jax import lax
from jax.experimental import pallas as pl
from jax.experimental.pallas import tpu as pltpu
from jax.experimental.pallas import tpu_sc as plsc

B, S, H = 64, 512, 768
L = 16
HJ = H // L
NW = 32
SW = S // NW
EPS = 1e-12
NBUF = 2


def _lanesum(x):
    lanes = lax.iota(jnp.int32, L)
    for k in (1, 2, 4, 8):
        perm = lanes ^ k
        x = x + x.at[perm].get(mode="promise_in_bounds")
    return x


def _rsqrt16(x):
    i = lax.bitcast_convert_type(x, jnp.int32)
    y = lax.bitcast_convert_type(jnp.int32(0x5F3759DF) - (i >> 1), jnp.float32)
    for _ in range(3):
        y = y * (1.5 - 0.5 * x * y * y)
    return y


def _sc_body(ids, wemb, pos, typ, gam, bet, out,
             idx_v, pt_v, ty_v, gb_v, buf0, buf1,
             sin0, sin1, sout0, sout1):
    c = lax.axis_index("c")
    s = lax.axis_index("s")
    w = s * 2 + c
    p0 = w * SW

    bufs = (buf0, buf1)
    sin = (sin0, sin1)
    sout = (sout0, sout1)

    pltpu.sync_copy(ids, idx_v)
    pltpu.sync_copy(pos.at[pl.ds(p0, SW), :], pt_v)
    pltpu.sync_copy(typ, ty_v)
    pltpu.sync_copy(gam, gb_v.at[0])
    pltpu.sync_copy(bet, gb_v.at[1])

    for r in range(SW):
        def _addty(j, _, r=r):
            off = j * L
            pt_v[r, pl.ds(off, L)] = pt_v[r, pl.ds(off, L)] + ty_v[0, pl.ds(off, L)]
            return 0
        lax.fori_loop(0, HJ, _addty, 0)

    def gather_start(b, k):
        pltpu.async_copy(wemb.at[idx_v.at[b, pl.ds(p0, SW)]], bufs[k], sin[k])

    def gather_wait(b, k):
        pltpu.make_async_copy(wemb.at[idx_v.at[b, pl.ds(p0, SW)]], bufs[k], sin[k]).wait()

    def out_start(b, k):
        pltpu.async_copy(bufs[k], out.at[b, pl.ds(p0, SW), :], sout[k])

    def out_wait(b, k):
        pltpu.make_async_copy(bufs[k], out.at[b, pl.ds(p0, SW), :], sout[k]).wait()

    def compute(k):
        buf = bufs[k]
        inv_h = jnp.float32(1.0 / H)
        for t in range(SW):
            def _stats(j, carry, t=t, buf=buf):
                acc, acc2 = carry
                off = j * L
                v = buf[t, pl.ds(off, L)] + pt_v[t, pl.ds(off, L)]
                buf[t, pl.ds(off, L)] = v
                return acc + v, acc2 + v * v
            z = jnp.zeros((L,), jnp.float32)
            acc, acc2 = lax.fori_loop(0, HJ, _stats, (z, z))
            m_vec = _lanesum(acc) * inv_h
            var = _lanesum(acc2) * inv_h - m_vec * m_vec
            r_vec = _rsqrt16(var + EPS)

            def _norm(j, _, t=t, buf=buf, r_vec=r_vec, m_vec=m_vec):
                off = j * L
                v = buf[t, pl.ds(off, L)]
                g = gb_v[0, pl.ds(off, L)]
                bb = gb_v[1, pl.ds(off, L)]
                buf[t, pl.ds(off, L)] = (v - m_vec) * r_vec * g + bb
                return 0
            lax.fori_loop(0, HJ, _norm, 0)

    for k in range(NBUF):
        gather_start(k, k)

    def loop_body(i, _):
        for k in range(NBUF):
            b = i * NBUF + k
            gather_wait(b, k)
            compute(k)
            out_start(b, k)

            @pl.when(b + NBUF < B)
            def _():
                out_wait(b, k)
                gather_start(b + NBUF, k)
        return 0

    lax.fori_loop(0, B // NBUF, loop_body, 0)

    for k in range(NBUF):
        out_wait(B - NBUF + k, k)


@functools.partial(jax.jit, static_argnums=())
def kernel(input_ids, word_emb, pos_emb, type_emb, gamma, beta):
    mesh = plsc.VectorSubcoreMesh(core_axis_name="c", subcore_axis_name="s")
    f = pl.kernel(
        _sc_body,
        mesh=mesh,
        out_type=jax.ShapeDtypeStruct((B, S, H), jnp.float32),
        scratch_types=[
            pltpu.VMEM((B, S), jnp.int32),
            pltpu.VMEM((SW, H), jnp.float32),
            pltpu.VMEM((2, H), jnp.float32),
            pltpu.VMEM((2, H), jnp.float32),
            pltpu.VMEM((SW, H), jnp.float32),
            pltpu.VMEM((SW, H), jnp.float32),
            pltpu.SemaphoreType.DMA,
            pltpu.SemaphoreType.DMA,
            pltpu.SemaphoreType.DMA,
            pltpu.SemaphoreType.DMA,
        ],
    )
    return f(input_ids.astype(jnp.int32), word_emb, pos_emb, type_emb, gamma, beta)


# speedup vs baseline: 1.0423x; 1.0423x over previous
import functools

import jax
import jax.numpy as jnp
from jax import lax
from jax.experimental import pallas as pl
from jax.experimental.pallas import tpu as pltpu
from jax.experimental.pallas import tpu_sc as plsc

B, S, H = 64, 512, 768
L = 16
HJ = H // L
NW = 32
SW = S // NW
EPS = 1e-12
NBUF = 2


def _lanesum(x):
    lanes = lax.iota(jnp.int32, L)
    for k in (1, 2, 4, 8):
        perm = lanes ^ k
        x = x + x.at[perm].get(mode="promise_in_bounds")
    return x


def _rsqrt16(x):
    i = lax.bitcast_convert_type(x, jnp.int32)
    y = lax.bitcast_convert_type(jnp.int32(0x5F3759DF) - (i >> 1), jnp.float32)
    for _ in range(3):
        y = y * (1.5 - 0.5 * x * y * y)
    return y


def _sc_body(ids, wemb, pos, typ, gam, bet, out,
             idx_v, pt_v, ty_v, gb_v, buf0, buf1,
             sin0, sin1, sout0, sout1):
    c = lax.axis_index("c")
    s = lax.axis_index("s")
    w = s * 2 + c
    p0 = w * SW

    bufs = (buf0, buf1)
    sin = (sin0, sin1)
    sout = (sout0, sout1)

    pltpu.sync_copy(ids, idx_v)
    pltpu.sync_copy(pos.at[pl.ds(p0, SW), :], pt_v)
    pltpu.sync_copy(typ, ty_v)
    pltpu.sync_copy(gam, gb_v.at[0])
    pltpu.sync_copy(bet, gb_v.at[1])

    for r in range(SW):
        def _addty(j, _, r=r):
            off = j * L
            pt_v[r, pl.ds(off, L)] = pt_v[r, pl.ds(off, L)] + ty_v[0, pl.ds(off, L)]
            return 0
        lax.fori_loop(0, HJ, _addty, 0)

    def gather_start(b, k):
        pltpu.async_copy(wemb.at[idx_v.at[b, pl.ds(p0, SW)]], bufs[k], sin[k])

    def gather_wait(b, k):
        pltpu.make_async_copy(wemb.at[idx_v.at[b, pl.ds(p0, SW)]], bufs[k], sin[k]).wait()

    def out_start(b, k):
        pltpu.async_copy(bufs[k], out.at[b, pl.ds(p0, SW), :], sout[k])

    def out_wait(b, k):
        pltpu.make_async_copy(bufs[k], out.at[b, pl.ds(p0, SW), :], sout[k]).wait()

    UNROLL = 8
    NITER = HJ // UNROLL

    def compute(k):
        buf = bufs[k]
        inv_h = jnp.float32(1.0 / H)

        def _token(t, _, buf=buf):
            def _stats(jj, carry, t=t, buf=buf):
                accs = list(carry)
                base = jj * (UNROLL * L)
                for u in range(UNROLL):
                    off = base + u * L
                    v = buf[t, pl.ds(off, L)] + pt_v[t, pl.ds(off, L)]
                    buf[t, pl.ds(off, L)] = v
                    accs[2 * (u % 2)] = accs[2 * (u % 2)] + v
                    accs[2 * (u % 2) + 1] = accs[2 * (u % 2) + 1] + v * v
                return tuple(accs)
            z = jnp.zeros((L,), jnp.float32)
            a0, s0, a1, s1 = lax.fori_loop(0, NITER, _stats, (z, z, z, z))
            m_vec = _lanesum(a0 + a1) * inv_h
            var = _lanesum(s0 + s1) * inv_h - m_vec * m_vec
            r_vec = _rsqrt16(var + EPS)

            def _norm(jj, _, t=t, buf=buf, r_vec=r_vec, m_vec=m_vec):
                base = jj * (UNROLL * L)
                for u in range(UNROLL):
                    off = base + u * L
                    v = buf[t, pl.ds(off, L)]
                    g = gb_v[0, pl.ds(off, L)]
                    bb = gb_v[1, pl.ds(off, L)]
                    buf[t, pl.ds(off, L)] = (v - m_vec) * r_vec * g + bb
                return 0
            lax.fori_loop(0, NITER, _norm, 0)
            return 0

        lax.fori_loop(0, SW, _token, 0)

    for k in range(NBUF):
        gather_start(k, k)

    def loop_body(i, _):
        for k in range(NBUF):
            b = i * NBUF + k
            gather_wait(b, k)
            compute(k)
            out_start(b, k)

            @pl.when(b + NBUF < B)
            def _():
                out_wait(b, k)
                gather_start(b + NBUF, k)
        return 0

    lax.fori_loop(0, B // NBUF, loop_body, 0)

    for k in range(NBUF):
        out_wait(B - NBUF + k, k)


@functools.partial(jax.jit, static_argnums=())
def kernel(input_ids, word_emb, pos_emb, type_emb, gamma, beta):
    mesh = plsc.VectorSubcoreMesh(core_axis_name="c", subcore_axis_name="s")
    f = pl.kernel(
        _sc_body,
        mesh=mesh,
        out_type=jax.ShapeDtypeStruct((B, S, H), jnp.float32),
        scratch_types=[
            pltpu.VMEM((B, S), jnp.int32),
            pltpu.VMEM((SW, H), jnp.float32),
            pltpu.VMEM((2, H), jnp.float32),
            pltpu.VMEM((2, H), jnp.float32),
            pltpu.VMEM((SW, H), jnp.float32),
            pltpu.VMEM((SW, H), jnp.float32),
            pltpu.SemaphoreType.DMA,
            pltpu.SemaphoreType.DMA,
            pltpu.SemaphoreType.DMA,
            pltpu.SemaphoreType.DMA,
        ],
    )
    return f(input_ids.astype(jnp.int32), word_emb, pos_emb, type_emb, gamma, beta)
